# Initial kernel scaffold; baseline (speedup 1.0000x reference)
import jax
import jax.numpy as jnp
from jax.experimental import pallas as pl


def kernel(hidden_states, router_weight, w13, w2, shared_gate_up, shared_down):
    raise NotImplementedError("write your pallas kernel here")


# speedup vs baseline: 1.1194x; 1.1194x over previous
import jax
import jax.numpy as jnp
from jax import lax
from jax.experimental import pallas as pl
from jax.experimental.pallas import tpu as pltpu
from jax.experimental.pallas import tpu_sc as plsc

_E = 8
_TOPK = 2
_H = 1024
_I = 2048
_IS = 2 * _I
_T = 2048
_NPAIR = _TOPK * _T
_BLK = 256
_NBLK = _NPAIR // _BLK + 8
_PPAD = _NBLK * _BLK
_SCW = 128
_QF = 4
_QH = _H // _QF
_ISB = 512


def _plan_body(x_ref, rw_ref, pos_ref, wcol_ref, bexp_ref, bsrc_ref, bval_ref):
    x = x_ref[...]
    rw = rw_ref[...]
    f32 = jnp.float32

    logits_t = lax.dot_general(rw, x, (((1,), (1,)), ((), ())),
                               preferred_element_type=f32)
    iota_e = lax.broadcasted_iota(jnp.int32, (_E, _T), 0)
    m1 = jnp.max(logits_t, axis=0, keepdims=True)
    e0 = jnp.min(jnp.where(logits_t == m1, iota_e, _E), axis=0, keepdims=True)
    masked = jnp.where(iota_e == e0, -1e30, logits_t)
    m2 = jnp.max(masked, axis=0, keepdims=True)
    e1 = jnp.min(jnp.where(masked == m2, iota_e, _E), axis=0, keepdims=True)

    mask_a = (iota_e == e0).astype(f32)
    mask_b = (iota_e == e1).astype(f32)
    masks = jnp.concatenate([mask_a, mask_b], axis=0)

    tr = lax.broadcasted_iota(jnp.int32, (_T, _T), 0)
    tc = lax.broadcasted_iota(jnp.int32, (_T, _T), 1)
    lt = (tr <= tc).astype(f32)
    pfx = lax.dot_general(masks, lt, (((1,), (0,)), ((), ())),
                          preferred_element_type=f32)
    pfx_a = pfx[0:_E]
    pfx_b = pfx[_E:2 * _E]
    end_a = lax.slice(pfx_a, (0, _T - 1), (_E, _T))
    end_b = lax.slice(pfx_b, (0, _T - 1), (_E, _T))
    n_e = end_a + end_b

    nblk = jnp.floor((n_e + (_BLK - 1.0)) * (1.0 / _BLK))
    sl_r = lax.broadcasted_iota(jnp.int32, (_E, _E), 0)
    sl_c = lax.broadcasted_iota(jnp.int32, (_E, _E), 1)
    sl = (sl_c < sl_r).astype(f32)
    blk_start = lax.dot_general(sl, nblk, (((1,), (0,)), ((), ())),
                                preferred_element_type=f32)
    used = lax.slice(blk_start + nblk, (_E - 1, 0), (_E, 1))
    pad_off = blk_start * float(_BLK)

    rank0 = jnp.sum(mask_a * pfx_a, axis=0, keepdims=True) - 1.0
    rank1 = (jnp.sum(mask_b * pfx_b, axis=0, keepdims=True) - 1.0
             + jnp.sum(mask_b * end_a, axis=0, keepdims=True))
    off0 = jnp.sum(mask_a * pad_off, axis=0, keepdims=True)
    off1 = jnp.sum(mask_b * pad_off, axis=0, keepdims=True)
    pos = jnp.concatenate([rank0 + off0, rank1 + off1], axis=0)
    pos_ref[...] = pos.astype(jnp.int32)

    ib = lax.broadcasted_iota(jnp.int32, (1, _NBLK), 1).astype(f32)
    in_blk = ((ib >= blk_start) & (ib < blk_start + nblk)).astype(f32)
    e_ids = lax.broadcasted_iota(jnp.int32, (_E, 1), 0).astype(f32)
    bexp = jnp.sum(e_ids * in_blk, axis=0, keepdims=True)
    ends_last = (nblk > 0.0) & (jnp.abs(blk_start + nblk - used) < 0.5)
    e_last = jnp.max(jnp.where(ends_last, e_ids, -1.0), axis=0, keepdims=True)
    valid = ib < used
    bexp_ref[...] = jnp.where(valid, bexp, e_last).astype(jnp.int32)
    bsrc_ref[...] = jnp.where(valid, ib, used - 1.0).astype(jnp.int32)
    bval_ref[...] = valid.astype(jnp.int32)

    logits_c = lax.dot_general(x, rw, (((1,), (1,)), ((), ())),
                               preferred_element_type=f32)
    iota_c = lax.broadcasted_iota(jnp.int32, (_T, _E), 1)
    m1c = jnp.max(logits_c, axis=1, keepdims=True)
    e0c = jnp.min(jnp.where(logits_c == m1c, iota_c, _E), axis=1, keepdims=True)
    m2c = jnp.max(jnp.where(iota_c == e0c, -1e30, logits_c), axis=1,
                  keepdims=True)
    w0 = 1.0 / (1.0 + jnp.exp(m2c - m1c))
    wcol_ref[...] = jnp.concatenate([w0, 1.0 - w0], axis=1)


def _plan(x, router_weight):
    return pl.pallas_call(
        _plan_body,
        out_shape=(
            jax.ShapeDtypeStruct((_TOPK, _T), jnp.int32),
            jax.ShapeDtypeStruct((_T, _TOPK), jnp.float32),
            jax.ShapeDtypeStruct((1, _NBLK), jnp.int32),
            jax.ShapeDtypeStruct((1, _NBLK), jnp.int32),
            jax.ShapeDtypeStruct((1, _NBLK), jnp.int32),
        ),
    )(x, router_weight)


def _sc_scatter(x, pos4):
    nidx = _NPAIR * _QF
    x4 = x.reshape(_T * _QF, _QH)
    mesh = plsc.VectorSubcoreMesh(core_axis_name="core",
                                  subcore_axis_name="subcore")

    @pl.kernel(out_type=jax.ShapeDtypeStruct((_PPAD * _QF, _QH), jnp.float32),
               mesh=mesh)
    def sck(x_hbm, i_hbm, o_hbm):
        def body(x_vmem, i_vmem):
            pltpu.sync_copy(x_vmem, o_hbm.at[i_vmem.at[0]])

        pltpu.emit_pipeline(
            body,
            grid=(nidx // _SCW,),
            in_specs=[
                pl.BlockSpec((_SCW, _QH),
                             lambda i: (i % (_T * _QF // _SCW), 0)),
                pl.BlockSpec((1, _SCW), lambda i: (0, i)),
            ],
            out_specs=[],
            core_axis_name="subcore",
            dimension_semantics=(pltpu.PARALLEL,),
        )(x_hbm, i_hbm)

    return sck(x4, pos4).reshape(_PPAD, _H)


def _sc_gather(ys, pos4):
    nidx = _NPAIR * _QF
    ys4 = ys.reshape(_PPAD * _QF, _QH)
    mesh = plsc.VectorSubcoreMesh(core_axis_name="core",
                                  subcore_axis_name="subcore")

    @pl.kernel(out_type=jax.ShapeDtypeStruct((nidx, _QH), jnp.float32),
               mesh=mesh)
    def gak(y_hbm, i_hbm, o_hbm):
        def body(i_vmem, o_vmem):
            pltpu.sync_copy(y_hbm.at[i_vmem.at[0]], o_vmem)

        pltpu.emit_pipeline(
            body,
            grid=(nidx // _SCW,),
            in_specs=[pl.BlockSpec((1, _SCW), lambda i: (0, i))],
            out_specs=[pl.BlockSpec((_SCW, _QH), lambda i: (i, 0))],
            core_axis_name="subcore",
            dimension_semantics=(pltpu.PARALLEL,),
        )(i_hbm, o_hbm)

    return gak(ys4, pos4).reshape(_NPAIR, _H)


def _grouped_body(bexp_s, bsrc_s, bval_s, xs_ref, w13_ref, w2_ref, ys_ref):
    i = pl.program_id(0)

    @pl.when(bval_s[i] == 1)
    def _():
        xb = xs_ref[...]
        wg = w13_ref[0, 0:_I, :]
        wu = w13_ref[0, _I:2 * _I, :]
        g = lax.dot_general(xb, wg, (((1,), (1,)), ((), ())),
                            preferred_element_type=jnp.float32)
        u = lax.dot_general(xb, wu, (((1,), (1,)), ((), ())),
                            preferred_element_type=jnp.float32)
        h = g * jax.nn.sigmoid(g) * u
        ys_ref[...] = lax.dot_general(h, w2_ref[0], (((1,), (1,)), ((), ())),
                                      preferred_element_type=jnp.float32)


def _grouped(bexp, bsrc, bval, xs, w13, w2):
    grid_spec = pltpu.PrefetchScalarGridSpec(
        num_scalar_prefetch=3,
        grid=(_NBLK,),
        in_specs=[
            pl.BlockSpec((_BLK, _H), lambda i, be, bs, bv: (bs[i], 0)),
            pl.BlockSpec((1, 2 * _I, _H), lambda i, be, bs, bv: (be[i], 0, 0)),
            pl.BlockSpec((1, _H, _I), lambda i, be, bs, bv: (be[i], 0, 0)),
        ],
        out_specs=pl.BlockSpec((_BLK, _H), lambda i, be, bs, bv: (i, 0)),
    )
    return pl.pallas_call(
        _grouped_body,
        grid_spec=grid_spec,
        out_shape=jax.ShapeDtypeStruct((_PPAD, _H), jnp.float32),
    )(bexp, bsrc, bval, xs, w13, w2)


def _shared_body(x_ref, wg_ref, wu_ref, wd_ref, out_ref):
    c = pl.program_id(0)
    xb = x_ref[...]
    g = lax.dot_general(xb, wg_ref[...], (((1,), (1,)), ((), ())),
                        preferred_element_type=jnp.float32)
    u = lax.dot_general(xb, wu_ref[...], (((1,), (1,)), ((), ())),
                        preferred_element_type=jnp.float32)
    h = g * jax.nn.sigmoid(g) * u
    y = lax.dot_general(h, wd_ref[...], (((1,), (1,)), ((), ())),
                        preferred_element_type=jnp.float32)

    @pl.when(c == 0)
    def _():
        out_ref[...] = y

    @pl.when(c > 0)
    def _():
        out_ref[...] += y


def _shared(x, shared_gate_up, shared_down):
    nchunk = _IS // _ISB
    return pl.pallas_call(
        _shared_body,
        grid=(nchunk,),
        in_specs=[
            pl.BlockSpec((_T, _H), lambda c: (0, 0)),
            pl.BlockSpec((_ISB, _H), lambda c: (c, 0)),
            pl.BlockSpec((_ISB, _H), lambda c: (c + nchunk, 0)),
            pl.BlockSpec((_H, _ISB), lambda c: (0, c)),
        ],
        out_specs=pl.BlockSpec((_T, _H), lambda c: (0, 0)),
        out_shape=jax.ShapeDtypeStruct((_T, _H), jnp.float32),
    )(x, shared_gate_up, shared_gate_up, shared_down)


def _epilogue_body(g0_ref, g1_ref, sh_ref, wc_ref, out_ref):
    wc = wc_ref[...]
    out_ref[...] = (wc[:, 0:1] * g0_ref[...] + wc[:, 1:2] * g1_ref[...]
                    + sh_ref[...])


def _epilogue(g0, g1, shared, wcol):
    nb = _T // _BLK
    return pl.pallas_call(
        _epilogue_body,
        grid=(nb,),
        in_specs=[
            pl.BlockSpec((_BLK, _H), lambda i: (i, 0)),
            pl.BlockSpec((_BLK, _H), lambda i: (i, 0)),
            pl.BlockSpec((_BLK, _H), lambda i: (i, 0)),
            pl.BlockSpec((_BLK, _TOPK), lambda i: (i, 0)),
        ],
        out_specs=pl.BlockSpec((_BLK, _H), lambda i: (i, 0)),
        out_shape=jax.ShapeDtypeStruct((_T, _H), jnp.float32),
    )(g0, g1, shared, wcol)


def kernel(hidden_states, router_weight, w13, w2, shared_gate_up, shared_down):
    b, s, h = hidden_states.shape
    x = hidden_states.reshape(_T, _H)
    pos, wcol, bexp, bsrc, bval = _plan(x, router_weight)
    pos4 = (pos.reshape(_NPAIR, 1) * _QF
            + jnp.arange(_QF, dtype=jnp.int32).reshape(1, _QF)
            ).reshape(1, _NPAIR * _QF)
    xs = _sc_scatter(x, pos4)
    ys = _grouped(bexp.reshape(_NBLK), bsrc.reshape(_NBLK),
                  bval.reshape(_NBLK), xs, w13, w2)
    shared = _shared(x, shared_gate_up, shared_down)
    gs = _sc_gather(ys, pos4)
    out = _epilogue(gs[0:_T], gs[_T:_NPAIR], shared, wcol)
    return out.reshape(b, s, h)
